# baseline (speedup 1.0000x reference)
import functools

import jax
import jax.numpy as jnp
from jax import lax
from jax.experimental import pallas as pl
from jax.experimental.pallas import tpu as pltpu
from jax.experimental.pallas import tpu_sc as plsc

B, C, H, W = 4, 96, 384, 384
CP = 128
S = B * H * W
N_PTS = 131072

NW = 32
CHUNK = 128
PER_W = N_PTS // NW
CHUNKS_PER_W = PER_W // CHUNK

ROWS_T = 32
BM = 2048

_sc_mesh = plsc.VectorSubcoreMesh(core_axis_name="c", subcore_axis_name="s")
_sc_params = pltpu.CompilerParams(use_tc_tiling_on_sc=False)


def _t_fwd_body(x_ref, z_ref):
    blk = x_ref[0]
    z_ref[:, :C] = jnp.transpose(blk.reshape(C, ROWS_T * W), (1, 0))
    z_ref[:, C:] = jnp.zeros((ROWS_T * W, CP - C), jnp.float32)


def _transpose_fwd(x):
    grid = (B, H // ROWS_T)
    return pl.pallas_call(
        _t_fwd_body,
        grid=grid,
        in_specs=[pl.BlockSpec((1, C, ROWS_T, W), lambda b, r: (b, 0, r, 0))],
        out_specs=pl.BlockSpec((ROWS_T * W, CP),
                               lambda b, r: (b * (H // ROWS_T) + r, 0)),
        out_shape=jax.ShapeDtypeStruct((S, CP), jnp.float32),
    )(x)


def _t_bwd_body(z_ref, o_ref):
    o_ref[0] = jnp.transpose(z_ref[:, :C], (1, 0)).reshape(C, ROWS_T, W)


def _transpose_bwd(z2d):
    grid = (B, H // ROWS_T)
    return pl.pallas_call(
        _t_bwd_body,
        grid=grid,
        in_specs=[pl.BlockSpec((ROWS_T * W, CP),
                               lambda b, r: (b * (H // ROWS_T) + r, 0))],
        out_specs=pl.BlockSpec((1, C, ROWS_T, W), lambda b, r: (b, 0, r, 0)),
        out_shape=jax.ShapeDtypeStruct((B, C, H, W), jnp.float32),
    )(z2d)


def _mm_body(td_ref, g_ref, wt_ref, b_ref, y_ref):
    g = g_ref[...]
    ghi = g.astype(jnp.bfloat16)
    glo = (g - ghi.astype(jnp.float32)).astype(jnp.bfloat16)
    wt = wt_ref[...]
    whi = wt.astype(jnp.bfloat16)
    wlo = (wt - whi.astype(jnp.float32)).astype(jnp.bfloat16)
    mm = jnp.dot(ghi, whi, preferred_element_type=jnp.float32)
    mm = mm + jnp.dot(glo, whi, preferred_element_type=jnp.float32)
    mm = mm + jnp.dot(ghi, wlo, preferred_element_type=jnp.float32)
    mm = mm + b_ref[...]
    tdf = td_ref[0]
    y_ref[...] = mm * tdf + g * (1.0 - tdf)


def _mm(tdf, g, wt, brow):
    grid = (N_PTS // BM,)
    return pl.pallas_call(
        _mm_body,
        grid=grid,
        in_specs=[
            pl.BlockSpec(memory_space=pltpu.SMEM),
            pl.BlockSpec((BM, CP), lambda i: (i, 0)),
            pl.BlockSpec((CP, CP), lambda i: (0, 0)),
            pl.BlockSpec((1, CP), lambda i: (0, 0)),
        ],
        out_specs=pl.BlockSpec((BM, CP), lambda i: (i, 0)),
        out_shape=jax.ShapeDtypeStruct((N_PTS, CP), jnp.float32),
    )(tdf, g, wt, brow)


SUPER = 256
N_SUPER = PER_W // SUPER
SPC = SUPER // CHUNK


@functools.partial(
    pl.kernel,
    out_type=jax.ShapeDtypeStruct((N_PTS, CP), jnp.float32),
    mesh=_sc_mesh,
    compiler_params=_sc_params,
    scratch_types=[
        pltpu.VMEM((CHUNKS_PER_W, CHUNK), jnp.int32),
        pltpu.VMEM((2, SUPER, CP), jnp.float32),
        pltpu.SemaphoreType.DMA,
        pltpu.SemaphoreType.DMA,
    ],
)
def _sc_gather(z_hbm, pos_hbm, g_hbm, idx_v, rows_v, gsem, wsem):
    wid = lax.axis_index("s") * 2 + lax.axis_index("c")
    c0 = wid * CHUNKS_PER_W
    pltpu.sync_copy(pos_hbm.at[pl.ds(c0, CHUNKS_PER_W)], idx_v)

    w_descs = [None] * N_SUPER
    for s in range(N_SUPER):
        p = s % 2
        if s >= 2:
            w_descs[s - 2].wait()
        g_descs = [
            pltpu.async_copy(
                z_hbm.at[idx_v.at[s * SPC + j]],
                rows_v.at[p, pl.ds(j * CHUNK, CHUNK)],
                gsem,
            )
            for j in range(SPC)
        ]
        for d in g_descs:
            d.wait()
        w_descs[s] = pltpu.async_copy(
            rows_v.at[p],
            g_hbm.at[pl.ds(wid * PER_W + s * SUPER, SUPER)],
            wsem,
        )
    for s in range(N_SUPER - 2, N_SUPER):
        w_descs[s].wait()


@functools.partial(
    pl.kernel,
    out_type=(),
    mesh=_sc_mesh,
    compiler_params=_sc_params,
    scratch_types=[
        pltpu.VMEM((CHUNKS_PER_W, CHUNK), jnp.int32),
        pltpu.VMEM((2, SUPER, CP), jnp.float32),
        pltpu.SemaphoreType.DMA,
        pltpu.SemaphoreType.DMA,
    ],
)
def _sc_scatter(y_hbm, pos_hbm, z_ref, idx_v, rows_v, rsem, ssem):
    wid = lax.axis_index("s") * 2 + lax.axis_index("c")
    c0 = wid * CHUNKS_PER_W
    pltpu.sync_copy(pos_hbm.at[pl.ds(c0, CHUNKS_PER_W)], idx_v)

    r_descs = [None] * N_SUPER
    s_descs = [None] * N_SUPER
    r_descs[0] = pltpu.async_copy(
        y_hbm.at[pl.ds(wid * PER_W, SUPER)], rows_v.at[0], rsem)
    for s in range(N_SUPER):
        p = s % 2
        if s + 1 < N_SUPER and s + 1 >= 2:
            for d in s_descs[s - 1]:
                d.wait()
        if s + 1 < N_SUPER:
            r_descs[s + 1] = pltpu.async_copy(
                y_hbm.at[pl.ds(wid * PER_W + (s + 1) * SUPER, SUPER)],
                rows_v.at[(s + 1) % 2],
                rsem,
            )
        r_descs[s].wait()
        s_descs[s] = [
            pltpu.async_copy(
                rows_v.at[p, pl.ds(j * CHUNK, CHUNK)],
                z_ref.at[idx_v.at[s * SPC + j]],
                ssem,
            )
            for j in range(SPC)
        ]
    for s in range(N_SUPER - 2, N_SUPER):
        for d in s_descs[s]:
            d.wait()


def kernel(x, indices, weight, bias, to_dense):
    pos = indices[:, 0] * (H * W) + indices[:, 1] * W + indices[:, 2]
    pos2d = pos.reshape(N_PTS // CHUNK, CHUNK)

    z2d = _transpose_fwd(x)
    g = _sc_gather(z2d, pos2d)

    tdf = jnp.where(to_dense, jnp.float32(1.0), jnp.float32(0.0)).reshape(1)
    wt_pad = jnp.zeros((CP, CP), jnp.float32).at[:C, :C].set(weight.T)
    b_pad = jnp.zeros((1, CP), jnp.float32).at[:, :C].set(bias.reshape(1, C))
    y = _mm(tdf, g, wt_pad, b_pad)

    z_ref = jax.new_ref(z2d)
    _sc_scatter(y, pos2d, z_ref)
    return _transpose_bwd(z_ref[...])


# speedup vs baseline: 1.0308x; 1.0308x over previous
import functools

import jax
import jax.numpy as jnp
from jax import lax
from jax.experimental import pallas as pl
from jax.experimental.pallas import tpu as pltpu
from jax.experimental.pallas import tpu_sc as plsc

B, C, H, W = 4, 96, 384, 384
CP = 128
S = B * H * W
N_PTS = 131072

NW = 32
CHUNK = 128
PER_W = N_PTS // NW
CHUNKS_PER_W = PER_W // CHUNK

ROWS_T = 64
BM = 2048

_sc_mesh = plsc.VectorSubcoreMesh(core_axis_name="c", subcore_axis_name="s")
_sc_params = pltpu.CompilerParams(use_tc_tiling_on_sc=False)


def _t_fwd_body(x_ref, z_ref):
    blk = x_ref[0]
    z_ref[:, :C] = jnp.transpose(blk.reshape(C, ROWS_T * W), (1, 0))
    z_ref[:, C:] = jnp.zeros((ROWS_T * W, CP - C), jnp.float32)


def _transpose_fwd(x):
    grid = (B, H // ROWS_T)
    return pl.pallas_call(
        _t_fwd_body,
        grid=grid,
        in_specs=[pl.BlockSpec((1, C, ROWS_T, W), lambda b, r: (b, 0, r, 0))],
        out_specs=pl.BlockSpec((ROWS_T * W, CP),
                               lambda b, r: (b * (H // ROWS_T) + r, 0)),
        out_shape=jax.ShapeDtypeStruct((S, CP), jnp.float32),
    )(x)


def _t_bwd_body(z_ref, o_ref):
    o_ref[0] = jnp.transpose(z_ref[:, :C], (1, 0)).reshape(C, ROWS_T, W)


def _transpose_bwd(z2d):
    grid = (B, H // ROWS_T)
    return pl.pallas_call(
        _t_bwd_body,
        grid=grid,
        in_specs=[pl.BlockSpec((ROWS_T * W, CP),
                               lambda b, r: (b * (H // ROWS_T) + r, 0))],
        out_specs=pl.BlockSpec((1, C, ROWS_T, W), lambda b, r: (b, 0, r, 0)),
        out_shape=jax.ShapeDtypeStruct((B, C, H, W), jnp.float32),
    )(z2d)


def _mm_body(td_ref, g_ref, wt_ref, b_ref, y_ref):
    g = g_ref[...]
    mm = jnp.dot(g, wt_ref[...], preferred_element_type=jnp.float32)
    mm = mm + b_ref[...]
    tdf = td_ref[0]
    y_ref[...] = mm * tdf + g * (1.0 - tdf)


def _mm(tdf, g, wt, brow):
    grid = (N_PTS // BM,)
    return pl.pallas_call(
        _mm_body,
        grid=grid,
        in_specs=[
            pl.BlockSpec(memory_space=pltpu.SMEM),
            pl.BlockSpec((BM, CP), lambda i: (i, 0)),
            pl.BlockSpec((CP, CP), lambda i: (0, 0)),
            pl.BlockSpec((1, CP), lambda i: (0, 0)),
        ],
        out_specs=pl.BlockSpec((BM, CP), lambda i: (i, 0)),
        out_shape=jax.ShapeDtypeStruct((N_PTS, CP), jnp.float32),
    )(tdf, g, wt, brow)


SUPER = 256
N_SUPER = PER_W // SUPER
SPC = SUPER // CHUNK


@functools.partial(
    pl.kernel,
    out_type=jax.ShapeDtypeStruct((N_PTS, CP), jnp.float32),
    mesh=_sc_mesh,
    compiler_params=_sc_params,
    scratch_types=[
        pltpu.VMEM((CHUNKS_PER_W, CHUNK), jnp.int32),
        pltpu.VMEM((2, SUPER, CP), jnp.float32),
        pltpu.SemaphoreType.DMA,
        pltpu.SemaphoreType.DMA,
    ],
)
def _sc_gather(z_hbm, pos_hbm, g_hbm, idx_v, rows_v, gsem, wsem):
    wid = lax.axis_index("s") * 2 + lax.axis_index("c")
    c0 = wid * CHUNKS_PER_W
    pltpu.sync_copy(pos_hbm.at[pl.ds(c0, CHUNKS_PER_W)], idx_v)

    w_descs = [None] * N_SUPER
    for s in range(N_SUPER):
        p = s % 2
        if s >= 2:
            w_descs[s - 2].wait()
        g_descs = [
            pltpu.async_copy(
                z_hbm.at[idx_v.at[s * SPC + j]],
                rows_v.at[p, pl.ds(j * CHUNK, CHUNK)],
                gsem,
            )
            for j in range(SPC)
        ]
        for d in g_descs:
            d.wait()
        w_descs[s] = pltpu.async_copy(
            rows_v.at[p],
            g_hbm.at[pl.ds(wid * PER_W + s * SUPER, SUPER)],
            wsem,
        )
    for s in range(N_SUPER - 2, N_SUPER):
        w_descs[s].wait()


@functools.partial(
    pl.kernel,
    out_type=(),
    mesh=_sc_mesh,
    compiler_params=_sc_params,
    scratch_types=[
        pltpu.VMEM((CHUNKS_PER_W, CHUNK), jnp.int32),
        pltpu.VMEM((2, SUPER, CP), jnp.float32),
        pltpu.SemaphoreType.DMA,
        pltpu.SemaphoreType.DMA,
    ],
)
def _sc_scatter(y_hbm, pos_hbm, z_ref, idx_v, rows_v, rsem, ssem):
    wid = lax.axis_index("s") * 2 + lax.axis_index("c")
    c0 = wid * CHUNKS_PER_W
    pltpu.sync_copy(pos_hbm.at[pl.ds(c0, CHUNKS_PER_W)], idx_v)

    r_descs = [None] * N_SUPER
    s_descs = [None] * N_SUPER
    r_descs[0] = pltpu.async_copy(
        y_hbm.at[pl.ds(wid * PER_W, SUPER)], rows_v.at[0], rsem)
    for s in range(N_SUPER):
        p = s % 2
        if s + 1 < N_SUPER and s + 1 >= 2:
            for d in s_descs[s - 1]:
                d.wait()
        if s + 1 < N_SUPER:
            r_descs[s + 1] = pltpu.async_copy(
                y_hbm.at[pl.ds(wid * PER_W + (s + 1) * SUPER, SUPER)],
                rows_v.at[(s + 1) % 2],
                rsem,
            )
        r_descs[s].wait()
        s_descs[s] = [
            pltpu.async_copy(
                rows_v.at[p, pl.ds(j * CHUNK, CHUNK)],
                z_ref.at[idx_v.at[s * SPC + j]],
                ssem,
            )
            for j in range(SPC)
        ]
    for s in range(N_SUPER - 2, N_SUPER):
        for d in s_descs[s]:
            d.wait()


def kernel(x, indices, weight, bias, to_dense):
    pos = indices[:, 0] * (H * W) + indices[:, 1] * W + indices[:, 2]
    pos2d = pos.reshape(N_PTS // CHUNK, CHUNK)

    z2d = _transpose_fwd(x)
    g = _sc_gather(z2d, pos2d)

    tdf = jnp.where(to_dense, jnp.float32(1.0), jnp.float32(0.0)).reshape(1)
    wt_pad = jnp.zeros((CP, CP), jnp.float32).at[:C, :C].set(weight.T)
    b_pad = jnp.zeros((1, CP), jnp.float32).at[:, :C].set(bias.reshape(1, C))
    y = _mm(tdf, g, wt_pad, b_pad)

    z_ref = jax.new_ref(z2d)
    _sc_scatter(y, pos2d, z_ref)
    return _transpose_bwd(z_ref[...])


# speedup vs baseline: 1.0392x; 1.0082x over previous
import functools

import jax
import jax.numpy as jnp
from jax import lax
from jax.experimental import pallas as pl
from jax.experimental.pallas import tpu as pltpu
from jax.experimental.pallas import tpu_sc as plsc

B, C, H, W = 4, 96, 384, 384
CP = 128
S = B * H * W
N_PTS = 131072

NW = 32
CHUNK = 128
PER_W = N_PTS // NW
CHUNKS_PER_W = PER_W // CHUNK

ROWS_T = 64
BM = 2048

_sc_mesh = plsc.VectorSubcoreMesh(core_axis_name="c", subcore_axis_name="s")
_sc_params = pltpu.CompilerParams(use_tc_tiling_on_sc=False)


def _t_fwd_body(x_ref, z_ref):
    blk = x_ref[0]
    z_ref[:, :C] = jnp.transpose(blk.reshape(C, ROWS_T * W), (1, 0))
    z_ref[:, C:] = jnp.zeros((ROWS_T * W, CP - C), jnp.float32)


def _transpose_fwd(x):
    grid = (B, H // ROWS_T)
    return pl.pallas_call(
        _t_fwd_body,
        grid=grid,
        in_specs=[pl.BlockSpec((1, C, ROWS_T, W), lambda b, r: (b, 0, r, 0))],
        out_specs=pl.BlockSpec((ROWS_T * W, CP),
                               lambda b, r: (b * (H // ROWS_T) + r, 0)),
        out_shape=jax.ShapeDtypeStruct((S, CP), jnp.float32),
    )(x)


def _t_bwd_body(z_ref, o_ref):
    o_ref[0] = jnp.transpose(z_ref[:, :C], (1, 0)).reshape(C, ROWS_T, W)


def _transpose_bwd(z2d):
    grid = (B, H // ROWS_T)
    return pl.pallas_call(
        _t_bwd_body,
        grid=grid,
        in_specs=[pl.BlockSpec((ROWS_T * W, CP),
                               lambda b, r: (b * (H // ROWS_T) + r, 0))],
        out_specs=pl.BlockSpec((1, C, ROWS_T, W), lambda b, r: (b, 0, r, 0)),
        out_shape=jax.ShapeDtypeStruct((B, C, H, W), jnp.float32),
    )(z2d)


def _mm_body(td_ref, g_ref, wt_ref, b_ref, y_ref):
    g = g_ref[...]
    mm = jnp.dot(g, wt_ref[...], preferred_element_type=jnp.float32)
    mm = mm + b_ref[...]
    tdf = td_ref[0]
    y_ref[...] = mm * tdf + g * (1.0 - tdf)


def _mm(tdf, g, wt, brow):
    grid = (N_PTS // BM,)
    return pl.pallas_call(
        _mm_body,
        grid=grid,
        in_specs=[
            pl.BlockSpec(memory_space=pltpu.SMEM),
            pl.BlockSpec((BM, CP), lambda i: (i, 0)),
            pl.BlockSpec((CP, CP), lambda i: (0, 0)),
            pl.BlockSpec((1, CP), lambda i: (0, 0)),
        ],
        out_specs=pl.BlockSpec((BM, CP), lambda i: (i, 0)),
        out_shape=jax.ShapeDtypeStruct((N_PTS, CP), jnp.float32),
    )(tdf, g, wt, brow)


SUPER = 256
SPC = SUPER // CHUNK
QSPLIT = 4
NQ = N_PTS // QSPLIT
PER_WQ = NQ // NW
CHUNKS_WQ = PER_WQ // CHUNK
N_SUPERQ = PER_WQ // SUPER


def _make_sc_gather(npts, per_w, chunks_w, n_super):
    @functools.partial(
        pl.kernel,
        out_type=jax.ShapeDtypeStruct((npts, CP), jnp.float32),
        mesh=_sc_mesh,
        compiler_params=_sc_params,
        scratch_types=[
            pltpu.VMEM((chunks_w, CHUNK), jnp.int32),
            pltpu.VMEM((2, SUPER, CP), jnp.float32),
            pltpu.SemaphoreType.DMA,
            pltpu.SemaphoreType.DMA,
        ],
    )
    def _gather(z_hbm, pos_hbm, g_hbm, idx_v, rows_v, gsem, wsem):
        wid = lax.axis_index("s") * 2 + lax.axis_index("c")
        c0 = wid * chunks_w
        pltpu.sync_copy(pos_hbm.at[pl.ds(c0, chunks_w)], idx_v)

        w_descs = [None] * n_super
        for s in range(n_super):
            p = s % 2
            if s >= 2:
                w_descs[s - 2].wait()
            g_descs = [
                pltpu.async_copy(
                    z_hbm.at[idx_v.at[s * SPC + j]],
                    rows_v.at[p, pl.ds(j * CHUNK, CHUNK)],
                    gsem,
                )
                for j in range(SPC)
            ]
            for d in g_descs:
                d.wait()
            w_descs[s] = pltpu.async_copy(
                rows_v.at[p],
                g_hbm.at[pl.ds(wid * per_w + s * SUPER, SUPER)],
                wsem,
            )
        for s in range(max(n_super - 2, 0), n_super):
            w_descs[s].wait()

    return _gather


def _make_sc_scatter(per_w, chunks_w, n_super):
    @functools.partial(
        pl.kernel,
        out_type=(),
        mesh=_sc_mesh,
        compiler_params=_sc_params,
        scratch_types=[
            pltpu.VMEM((chunks_w, CHUNK), jnp.int32),
            pltpu.VMEM((2, SUPER, CP), jnp.float32),
            pltpu.SemaphoreType.DMA,
            pltpu.SemaphoreType.DMA,
        ],
    )
    def _scatter(y_hbm, pos_hbm, z_ref, idx_v, rows_v, rsem, ssem):
        wid = lax.axis_index("s") * 2 + lax.axis_index("c")
        c0 = wid * chunks_w
        pltpu.sync_copy(pos_hbm.at[pl.ds(c0, chunks_w)], idx_v)

        r_descs = [None] * n_super
        s_descs = [None] * n_super
        r_descs[0] = pltpu.async_copy(
            y_hbm.at[pl.ds(wid * per_w, SUPER)], rows_v.at[0], rsem)
        for s in range(n_super):
            p = s % 2
            if s + 1 < n_super and s + 1 >= 2:
                for d in s_descs[s - 1]:
                    d.wait()
            if s + 1 < n_super:
                r_descs[s + 1] = pltpu.async_copy(
                    y_hbm.at[pl.ds(wid * per_w + (s + 1) * SUPER, SUPER)],
                    rows_v.at[(s + 1) % 2],
                    rsem,
                )
            r_descs[s].wait()
            s_descs[s] = [
                pltpu.async_copy(
                    rows_v.at[p, pl.ds(j * CHUNK, CHUNK)],
                    z_ref.at[idx_v.at[s * SPC + j]],
                    ssem,
                )
                for j in range(SPC)
            ]
        for s in range(max(n_super - 2, 0), n_super):
            for d in s_descs[s]:
                d.wait()

    return _scatter


_sc_gather_q = _make_sc_gather(NQ, PER_WQ, CHUNKS_WQ, N_SUPERQ)
_sc_scatter_q = _make_sc_scatter(PER_WQ, CHUNKS_WQ, N_SUPERQ)


def _mm_q(tdf, g, wt, brow):
    grid = (NQ // BM,)
    return pl.pallas_call(
        _mm_body,
        grid=grid,
        in_specs=[
            pl.BlockSpec(memory_space=pltpu.SMEM),
            pl.BlockSpec((BM, CP), lambda i: (i, 0)),
            pl.BlockSpec((CP, CP), lambda i: (0, 0)),
            pl.BlockSpec((1, CP), lambda i: (0, 0)),
        ],
        out_specs=pl.BlockSpec((BM, CP), lambda i: (i, 0)),
        out_shape=jax.ShapeDtypeStruct((NQ, CP), jnp.float32),
    )(tdf, g, wt, brow)


def kernel(x, indices, weight, bias, to_dense):
    pos = indices[:, 0] * (H * W) + indices[:, 1] * W + indices[:, 2]
    pos2d = pos.reshape(N_PTS // CHUNK, CHUNK)
    rows_q = (N_PTS // CHUNK) // QSPLIT

    z2d = _transpose_fwd(x)

    tdf = jnp.where(to_dense, jnp.float32(1.0), jnp.float32(0.0)).reshape(1)
    wt_pad = jnp.zeros((CP, CP), jnp.float32).at[:C, :C].set(weight.T)
    b_pad = jnp.zeros((1, CP), jnp.float32).at[:, :C].set(bias.reshape(1, C))

    pos_q = [pos2d[q * rows_q:(q + 1) * rows_q] for q in range(QSPLIT)]
    g_q = [_sc_gather_q(z2d, pos_q[q]) for q in range(QSPLIT)]
    y_q = [_mm_q(tdf, g_q[q], wt_pad, b_pad) for q in range(QSPLIT)]

    z_ref = jax.new_ref(z2d)
    for q in range(QSPLIT):
        _sc_scatter_q(y_q[q], pos_q[q], z_ref)
    return _transpose_bwd(z_ref[...])


# speedup vs baseline: 1.0637x; 1.0236x over previous
import functools

import jax
import jax.numpy as jnp
from jax import lax
from jax.experimental import pallas as pl
from jax.experimental.pallas import tpu as pltpu
from jax.experimental.pallas import tpu_sc as plsc

B, C, H, W = 4, 96, 384, 384
CP = 128
S = B * H * W
N_PTS = 131072

NW = 32
CHUNK = 128
PER_W = N_PTS // NW
CHUNKS_PER_W = PER_W // CHUNK

ROWS_T = 64
BM = 2048

_sc_mesh = plsc.VectorSubcoreMesh(core_axis_name="c", subcore_axis_name="s")
_sc_params = pltpu.CompilerParams(use_tc_tiling_on_sc=False)


def _t_fwd_body(x_ref, z_ref):
    blk = x_ref[0]
    z_ref[:, :C] = jnp.transpose(blk.reshape(C, ROWS_T * W), (1, 0))
    z_ref[:, C:] = jnp.zeros((ROWS_T * W, CP - C), jnp.float32)


def _transpose_fwd(x):
    grid = (B, H // ROWS_T)
    return pl.pallas_call(
        _t_fwd_body,
        grid=grid,
        in_specs=[pl.BlockSpec((1, C, ROWS_T, W), lambda b, r: (b, 0, r, 0))],
        out_specs=pl.BlockSpec((ROWS_T * W, CP),
                               lambda b, r: (b * (H // ROWS_T) + r, 0)),
        out_shape=jax.ShapeDtypeStruct((S, CP), jnp.float32),
    )(x)


def _t_bwd_body(z_ref, o_ref):
    o_ref[0] = jnp.transpose(z_ref[:, :C], (1, 0)).reshape(C, ROWS_T, W)


def _transpose_bwd(z2d):
    grid = (B, H // ROWS_T)
    return pl.pallas_call(
        _t_bwd_body,
        grid=grid,
        in_specs=[pl.BlockSpec((ROWS_T * W, CP),
                               lambda b, r: (b * (H // ROWS_T) + r, 0))],
        out_specs=pl.BlockSpec((1, C, ROWS_T, W), lambda b, r: (b, 0, r, 0)),
        out_shape=jax.ShapeDtypeStruct((B, C, H, W), jnp.float32),
    )(z2d)


def _mm_body(td_ref, g_ref, wt_ref, b_ref, y_ref):
    g = g_ref[...]
    mm = jnp.dot(g, wt_ref[...], preferred_element_type=jnp.float32)
    mm = mm + b_ref[...]
    tdf = td_ref[0]
    y_ref[...] = mm * tdf + g * (1.0 - tdf)


def _mm(tdf, g, wt, brow):
    grid = (N_PTS // BM,)
    return pl.pallas_call(
        _mm_body,
        grid=grid,
        in_specs=[
            pl.BlockSpec(memory_space=pltpu.SMEM),
            pl.BlockSpec((BM, CP), lambda i: (i, 0)),
            pl.BlockSpec((CP, CP), lambda i: (0, 0)),
            pl.BlockSpec((1, CP), lambda i: (0, 0)),
        ],
        out_specs=pl.BlockSpec((BM, CP), lambda i: (i, 0)),
        out_shape=jax.ShapeDtypeStruct((N_PTS, CP), jnp.float32),
    )(tdf, g, wt, brow)


SUPER = 256
SPC = SUPER // CHUNK
QSPLIT = 2
NQ = N_PTS // QSPLIT
PER_WQ = NQ // NW
CHUNKS_WQ = PER_WQ // CHUNK
N_SUPERQ = PER_WQ // SUPER


def _make_sc_gather(npts, per_w, chunks_w, n_super):
    @functools.partial(
        pl.kernel,
        out_type=jax.ShapeDtypeStruct((npts, CP), jnp.float32),
        mesh=_sc_mesh,
        compiler_params=_sc_params,
        scratch_types=[
            pltpu.VMEM((chunks_w, CHUNK), jnp.int32),
            pltpu.VMEM((2, SUPER, CP), jnp.float32),
            pltpu.SemaphoreType.DMA,
            pltpu.SemaphoreType.DMA,
        ],
    )
    def _gather(z_hbm, pos_hbm, g_hbm, idx_v, rows_v, gsem, wsem):
        wid = lax.axis_index("s") * 2 + lax.axis_index("c")
        c0 = wid * chunks_w
        pltpu.sync_copy(pos_hbm.at[pl.ds(c0, chunks_w)], idx_v)

        w_descs = [None] * n_super
        for s in range(n_super):
            p = s % 2
            if s >= 2:
                w_descs[s - 2].wait()
            g_descs = [
                pltpu.async_copy(
                    z_hbm.at[idx_v.at[s * SPC + j]],
                    rows_v.at[p, pl.ds(j * CHUNK, CHUNK)],
                    gsem,
                )
                for j in range(SPC)
            ]
            for d in g_descs:
                d.wait()
            w_descs[s] = pltpu.async_copy(
                rows_v.at[p],
                g_hbm.at[pl.ds(wid * per_w + s * SUPER, SUPER)],
                wsem,
            )
        for s in range(max(n_super - 2, 0), n_super):
            w_descs[s].wait()

    return _gather


def _make_sc_scatter(per_w, chunks_w, n_super):
    @functools.partial(
        pl.kernel,
        out_type=(),
        mesh=_sc_mesh,
        compiler_params=_sc_params,
        scratch_types=[
            pltpu.VMEM((chunks_w, CHUNK), jnp.int32),
            pltpu.VMEM((2, SUPER, CP), jnp.float32),
            pltpu.SemaphoreType.DMA,
            pltpu.SemaphoreType.DMA,
        ],
    )
    def _scatter(y_hbm, pos_hbm, z_ref, idx_v, rows_v, rsem, ssem):
        wid = lax.axis_index("s") * 2 + lax.axis_index("c")
        c0 = wid * chunks_w
        pltpu.sync_copy(pos_hbm.at[pl.ds(c0, chunks_w)], idx_v)

        r_descs = [None] * n_super
        s_descs = [None] * n_super
        r_descs[0] = pltpu.async_copy(
            y_hbm.at[pl.ds(wid * per_w, SUPER)], rows_v.at[0], rsem)
        for s in range(n_super):
            p = s % 2
            if s + 1 < n_super and s + 1 >= 2:
                for d in s_descs[s - 1]:
                    d.wait()
            if s + 1 < n_super:
                r_descs[s + 1] = pltpu.async_copy(
                    y_hbm.at[pl.ds(wid * per_w + (s + 1) * SUPER, SUPER)],
                    rows_v.at[(s + 1) % 2],
                    rsem,
                )
            r_descs[s].wait()
            s_descs[s] = [
                pltpu.async_copy(
                    rows_v.at[p, pl.ds(j * CHUNK, CHUNK)],
                    z_ref.at[idx_v.at[s * SPC + j]],
                    ssem,
                )
                for j in range(SPC)
            ]
        for s in range(max(n_super - 2, 0), n_super):
            for d in s_descs[s]:
                d.wait()

    return _scatter


_sc_gather_q = _make_sc_gather(NQ, PER_WQ, CHUNKS_WQ, N_SUPERQ)
_sc_scatter_q = _make_sc_scatter(PER_WQ, CHUNKS_WQ, N_SUPERQ)


def _mm_q(tdf, g, wt, brow):
    grid = (NQ // BM,)
    return pl.pallas_call(
        _mm_body,
        grid=grid,
        in_specs=[
            pl.BlockSpec(memory_space=pltpu.SMEM),
            pl.BlockSpec((BM, CP), lambda i: (i, 0)),
            pl.BlockSpec((CP, CP), lambda i: (0, 0)),
            pl.BlockSpec((1, CP), lambda i: (0, 0)),
        ],
        out_specs=pl.BlockSpec((BM, CP), lambda i: (i, 0)),
        out_shape=jax.ShapeDtypeStruct((NQ, CP), jnp.float32),
    )(tdf, g, wt, brow)


def kernel(x, indices, weight, bias, to_dense):
    pos = indices[:, 0] * (H * W) + indices[:, 1] * W + indices[:, 2]
    pos2d = pos.reshape(N_PTS // CHUNK, CHUNK)
    rows_q = (N_PTS // CHUNK) // QSPLIT

    z2d = _transpose_fwd(x)

    tdf = jnp.where(to_dense, jnp.float32(1.0), jnp.float32(0.0)).reshape(1)
    wt_pad = jnp.zeros((CP, CP), jnp.float32).at[:C, :C].set(weight.T)
    b_pad = jnp.zeros((1, CP), jnp.float32).at[:, :C].set(bias.reshape(1, C))

    pos_q = [pos2d[q * rows_q:(q + 1) * rows_q] for q in range(QSPLIT)]
    g_q = [_sc_gather_q(z2d, pos_q[q]) for q in range(QSPLIT)]
    y_q = [_mm_q(tdf, g_q[q], wt_pad, b_pad) for q in range(QSPLIT)]

    z_ref = jax.new_ref(z2d)
    for q in range(QSPLIT):
        _sc_scatter_q(y_q[q], pos_q[q], z_ref)
    return _transpose_bwd(z_ref[...])


# speedup vs baseline: 1.0715x; 1.0073x over previous
import functools

import jax
import jax.numpy as jnp
from jax import lax
from jax.experimental import pallas as pl
from jax.experimental.pallas import tpu as pltpu
from jax.experimental.pallas import tpu_sc as plsc

B, C, H, W = 4, 96, 384, 384
CP = 128
S = B * H * W
N_PTS = 131072

NW = 32
CHUNK = 128
PER_W = N_PTS // NW
CHUNKS_PER_W = PER_W // CHUNK

ROWS_T = 64
BM = 2048

_sc_mesh = plsc.VectorSubcoreMesh(core_axis_name="c", subcore_axis_name="s")
_sc_params = pltpu.CompilerParams(use_tc_tiling_on_sc=False)


def _t_fwd_body(x_ref, z_ref):
    blk = x_ref[0]
    z_ref[:, :C] = jnp.transpose(blk.reshape(C, ROWS_T * W), (1, 0))
    z_ref[:, C:] = jnp.zeros((ROWS_T * W, CP - C), jnp.float32)


def _transpose_fwd(x):
    grid = (B, H // ROWS_T)
    return pl.pallas_call(
        _t_fwd_body,
        grid=grid,
        in_specs=[pl.BlockSpec((1, C, ROWS_T, W), lambda b, r: (b, 0, r, 0))],
        out_specs=pl.BlockSpec((ROWS_T * W, CP),
                               lambda b, r: (b * (H // ROWS_T) + r, 0)),
        out_shape=jax.ShapeDtypeStruct((S, CP), jnp.float32),
    )(x)


def _t_bwd_body(z_ref, o_ref):
    o_ref[0] = jnp.transpose(z_ref[:, :C], (1, 0)).reshape(C, ROWS_T, W)


def _transpose_bwd(z2d):
    grid = (B, H // ROWS_T)
    return pl.pallas_call(
        _t_bwd_body,
        grid=grid,
        in_specs=[pl.BlockSpec((ROWS_T * W, CP),
                               lambda b, r: (b * (H // ROWS_T) + r, 0))],
        out_specs=pl.BlockSpec((1, C, ROWS_T, W), lambda b, r: (b, 0, r, 0)),
        out_shape=jax.ShapeDtypeStruct((B, C, H, W), jnp.float32),
    )(z2d)


def _mm_body(td_ref, g_ref, wt_ref, b_ref, y_ref):
    g = g_ref[...]
    mm = jnp.dot(g, wt_ref[...], preferred_element_type=jnp.float32)
    mm = mm + b_ref[...]
    tdf = td_ref[0]
    y_ref[...] = mm * tdf + g * (1.0 - tdf)


def _mm(tdf, g, wt, brow):
    grid = (N_PTS // BM,)
    return pl.pallas_call(
        _mm_body,
        grid=grid,
        in_specs=[
            pl.BlockSpec(memory_space=pltpu.SMEM),
            pl.BlockSpec((BM, CP), lambda i: (i, 0)),
            pl.BlockSpec((CP, CP), lambda i: (0, 0)),
            pl.BlockSpec((1, CP), lambda i: (0, 0)),
        ],
        out_specs=pl.BlockSpec((BM, CP), lambda i: (i, 0)),
        out_shape=jax.ShapeDtypeStruct((N_PTS, CP), jnp.float32),
    )(tdf, g, wt, brow)


SUPER = 256
SPC = SUPER // CHUNK
QSPLIT = 2
NQ = N_PTS // QSPLIT
PER_WQ = NQ // NW
CHUNKS_WQ = PER_WQ // CHUNK
N_SUPERQ = PER_WQ // SUPER


def _make_sc_gather(npts, per_w, chunks_w, n_super):
    @functools.partial(
        pl.kernel,
        out_type=jax.ShapeDtypeStruct((npts, CP), jnp.float32),
        mesh=_sc_mesh,
        compiler_params=_sc_params,
        scratch_types=[
            pltpu.VMEM((chunks_w, CHUNK), jnp.int32),
            pltpu.VMEM((2, SUPER, CP), jnp.float32),
            pltpu.SemaphoreType.DMA,
            pltpu.SemaphoreType.DMA,
        ],
    )
    def _gather(z_hbm, pos_hbm, g_hbm, idx_v, rows_v, gsem, wsem):
        wid = lax.axis_index("s") * 2 + lax.axis_index("c")
        c0 = wid * chunks_w
        pltpu.sync_copy(pos_hbm.at[pl.ds(c0, chunks_w)], idx_v)

        w_descs = [None] * n_super
        g_descs = [None] * n_super
        for s in range(n_super):
            p = s % 2
            if s >= 2:
                w_descs[s - 2].wait()
            g_descs[s] = [
                pltpu.async_copy(
                    z_hbm.at[idx_v.at[s * SPC + j]],
                    rows_v.at[p, pl.ds(j * CHUNK, CHUNK)],
                    gsem,
                )
                for j in range(SPC)
            ]
            if s >= 1:
                for d in g_descs[s - 1]:
                    d.wait()
                w_descs[s - 1] = pltpu.async_copy(
                    rows_v.at[(s - 1) % 2],
                    g_hbm.at[pl.ds(wid * per_w + (s - 1) * SUPER, SUPER)],
                    wsem,
                )
        for d in g_descs[n_super - 1]:
            d.wait()
        w_descs[n_super - 1] = pltpu.async_copy(
            rows_v.at[(n_super - 1) % 2],
            g_hbm.at[pl.ds(wid * per_w + (n_super - 1) * SUPER, SUPER)],
            wsem,
        )
        for s in range(max(n_super - 2, 0), n_super):
            w_descs[s].wait()

    return _gather


def _make_sc_scatter(per_w, chunks_w, n_super):
    @functools.partial(
        pl.kernel,
        out_type=(),
        mesh=_sc_mesh,
        compiler_params=_sc_params,
        scratch_types=[
            pltpu.VMEM((chunks_w, CHUNK), jnp.int32),
            pltpu.VMEM((2, SUPER, CP), jnp.float32),
            pltpu.SemaphoreType.DMA,
            pltpu.SemaphoreType.DMA,
        ],
    )
    def _scatter(y_hbm, pos_hbm, z_ref, idx_v, rows_v, rsem, ssem):
        wid = lax.axis_index("s") * 2 + lax.axis_index("c")
        c0 = wid * chunks_w
        pltpu.sync_copy(pos_hbm.at[pl.ds(c0, chunks_w)], idx_v)

        r_descs = [None] * n_super
        s_descs = [None] * n_super
        r_descs[0] = pltpu.async_copy(
            y_hbm.at[pl.ds(wid * per_w, SUPER)], rows_v.at[0], rsem)
        for s in range(n_super):
            p = s % 2
            if s + 1 < n_super and s + 1 >= 2:
                for d in s_descs[s - 1]:
                    d.wait()
            if s + 1 < n_super:
                r_descs[s + 1] = pltpu.async_copy(
                    y_hbm.at[pl.ds(wid * per_w + (s + 1) * SUPER, SUPER)],
                    rows_v.at[(s + 1) % 2],
                    rsem,
                )
            r_descs[s].wait()
            s_descs[s] = [
                pltpu.async_copy(
                    rows_v.at[p, pl.ds(j * CHUNK, CHUNK)],
                    z_ref.at[idx_v.at[s * SPC + j]],
                    ssem,
                )
                for j in range(SPC)
            ]
        for s in range(max(n_super - 2, 0), n_super):
            for d in s_descs[s]:
                d.wait()

    return _scatter


_sc_gather_q = _make_sc_gather(NQ, PER_WQ, CHUNKS_WQ, N_SUPERQ)
_sc_scatter_q = _make_sc_scatter(PER_WQ, CHUNKS_WQ, N_SUPERQ)


def _mm_q(tdf, g, wt, brow):
    grid = (NQ // BM,)
    return pl.pallas_call(
        _mm_body,
        grid=grid,
        in_specs=[
            pl.BlockSpec(memory_space=pltpu.SMEM),
            pl.BlockSpec((BM, CP), lambda i: (i, 0)),
            pl.BlockSpec((CP, CP), lambda i: (0, 0)),
            pl.BlockSpec((1, CP), lambda i: (0, 0)),
        ],
        out_specs=pl.BlockSpec((BM, CP), lambda i: (i, 0)),
        out_shape=jax.ShapeDtypeStruct((NQ, CP), jnp.float32),
    )(tdf, g, wt, brow)


def kernel(x, indices, weight, bias, to_dense):
    pos = indices[:, 0] * (H * W) + indices[:, 1] * W + indices[:, 2]
    pos2d = pos.reshape(N_PTS // CHUNK, CHUNK)
    rows_q = (N_PTS // CHUNK) // QSPLIT

    z2d = _transpose_fwd(x)

    tdf = jnp.where(to_dense, jnp.float32(1.0), jnp.float32(0.0)).reshape(1)
    wt_pad = jnp.zeros((CP, CP), jnp.float32).at[:C, :C].set(weight.T)
    b_pad = jnp.zeros((1, CP), jnp.float32).at[:, :C].set(bias.reshape(1, C))

    pos_q = [pos2d[q * rows_q:(q + 1) * rows_q] for q in range(QSPLIT)]
    g_q = [_sc_gather_q(z2d, pos_q[q]) for q in range(QSPLIT)]
    y_q = [_mm_q(tdf, g_q[q], wt_pad, b_pad) for q in range(QSPLIT)]

    z_ref = jax.new_ref(z2d)
    for q in range(QSPLIT):
        _sc_scatter_q(y_q[q], pos_q[q], z_ref)
    return _transpose_bwd(z_ref[...])


# speedup vs baseline: 1.0718x; 1.0003x over previous
import functools

import jax
import jax.numpy as jnp
from jax import lax
from jax.experimental import pallas as pl
from jax.experimental.pallas import tpu as pltpu
from jax.experimental.pallas import tpu_sc as plsc

B, C, H, W = 4, 96, 384, 384
CP = 128
S = B * H * W
N_PTS = 131072

NW = 32
CHUNK = 128
PER_W = N_PTS // NW
CHUNKS_PER_W = PER_W // CHUNK

ROWS_T = 64
BM = 2048

_sc_mesh = plsc.VectorSubcoreMesh(core_axis_name="c", subcore_axis_name="s")
_sc_params = pltpu.CompilerParams(use_tc_tiling_on_sc=False)


def _t_fwd_body(x_ref, z_ref):
    blk = x_ref[0]
    z_ref[:, :C] = jnp.transpose(blk.reshape(C, ROWS_T * W), (1, 0))
    z_ref[:, C:] = jnp.zeros((ROWS_T * W, CP - C), jnp.float32)


def _transpose_fwd(x):
    grid = (B, H // ROWS_T)
    return pl.pallas_call(
        _t_fwd_body,
        grid=grid,
        in_specs=[pl.BlockSpec((1, C, ROWS_T, W), lambda b, r: (b, 0, r, 0))],
        out_specs=pl.BlockSpec((ROWS_T * W, CP),
                               lambda b, r: (b * (H // ROWS_T) + r, 0)),
        out_shape=jax.ShapeDtypeStruct((S, CP), jnp.float32),
    )(x)


def _t_bwd_body(z_ref, o_ref):
    o_ref[0] = jnp.transpose(z_ref[:, :C], (1, 0)).reshape(C, ROWS_T, W)


def _transpose_bwd(z2d):
    grid = (B, H // ROWS_T)
    return pl.pallas_call(
        _t_bwd_body,
        grid=grid,
        in_specs=[pl.BlockSpec((ROWS_T * W, CP),
                               lambda b, r: (b * (H // ROWS_T) + r, 0))],
        out_specs=pl.BlockSpec((1, C, ROWS_T, W), lambda b, r: (b, 0, r, 0)),
        out_shape=jax.ShapeDtypeStruct((B, C, H, W), jnp.float32),
    )(z2d)


def _mm_body(td_ref, g_ref, wt_ref, b_ref, y_ref):
    g = g_ref[...]
    mm = jnp.dot(g, wt_ref[...], preferred_element_type=jnp.float32)
    mm = mm + b_ref[...]
    tdf = td_ref[0]
    y_ref[...] = mm * tdf + g * (1.0 - tdf)


SUPER = 256
SPC = SUPER // CHUNK
QSPLIT = 2
NQ = N_PTS // QSPLIT
PER_WQ = NQ // NW
CHUNKS_WQ = PER_WQ // CHUNK
N_SUPERQ = PER_WQ // SUPER


def _make_sc_gather(npts, per_w, chunks_w, n_super):
    @functools.partial(
        pl.kernel,
        out_type=jax.ShapeDtypeStruct((npts, CP), jnp.float32),
        mesh=_sc_mesh,
        compiler_params=_sc_params,
        scratch_types=[
            pltpu.VMEM((chunks_w, CHUNK), jnp.int32),
            pltpu.VMEM((2, SUPER, CP), jnp.float32),
            pltpu.SemaphoreType.DMA,
            pltpu.SemaphoreType.DMA,
        ],
    )
    def _gather(z_hbm, pos_hbm, g_hbm, idx_v, rows_v, gsem, wsem):
        wid = lax.axis_index("s") * 2 + lax.axis_index("c")
        c0 = wid * chunks_w
        pltpu.sync_copy(pos_hbm.at[pl.ds(c0, chunks_w)], idx_v)

        w_descs = [None] * n_super
        g_descs = [None] * n_super
        for s in range(n_super):
            p = s % 2
            if s >= 2:
                w_descs[s - 2].wait()
            g_descs[s] = [
                pltpu.async_copy(
                    z_hbm.at[idx_v.at[s * SPC + j]],
                    rows_v.at[p, pl.ds(j * CHUNK, CHUNK)],
                    gsem,
                )
                for j in range(SPC)
            ]
            if s >= 1:
                for d in g_descs[s - 1]:
                    d.wait()
                w_descs[s - 1] = pltpu.async_copy(
                    rows_v.at[(s - 1) % 2],
                    g_hbm.at[pl.ds(wid * per_w + (s - 1) * SUPER, SUPER)],
                    wsem,
                )
        for d in g_descs[n_super - 1]:
            d.wait()
        w_descs[n_super - 1] = pltpu.async_copy(
            rows_v.at[(n_super - 1) % 2],
            g_hbm.at[pl.ds(wid * per_w + (n_super - 1) * SUPER, SUPER)],
            wsem,
        )
        for s in range(max(n_super - 2, 0), n_super):
            w_descs[s].wait()

    return _gather


def _make_sc_scatter(per_w, chunks_w, n_super):
    @functools.partial(
        pl.kernel,
        out_type=(),
        mesh=_sc_mesh,
        compiler_params=_sc_params,
        scratch_types=[
            pltpu.VMEM((chunks_w, CHUNK), jnp.int32),
            pltpu.VMEM((2, SUPER, CP), jnp.float32),
            pltpu.SemaphoreType.DMA,
            pltpu.SemaphoreType.DMA,
        ],
    )
    def _scatter(y_hbm, pos_hbm, z_ref, idx_v, rows_v, rsem, ssem):
        wid = lax.axis_index("s") * 2 + lax.axis_index("c")
        c0 = wid * chunks_w
        pltpu.sync_copy(pos_hbm.at[pl.ds(c0, chunks_w)], idx_v)

        r_descs = [None] * n_super
        s_descs = [None] * n_super
        r_descs[0] = pltpu.async_copy(
            y_hbm.at[pl.ds(wid * per_w, SUPER)], rows_v.at[0], rsem)
        for s in range(n_super):
            p = s % 2
            if s + 1 < n_super and s + 1 >= 2:
                for d in s_descs[s - 1]:
                    d.wait()
            if s + 1 < n_super:
                r_descs[s + 1] = pltpu.async_copy(
                    y_hbm.at[pl.ds(wid * per_w + (s + 1) * SUPER, SUPER)],
                    rows_v.at[(s + 1) % 2],
                    rsem,
                )
            r_descs[s].wait()
            s_descs[s] = [
                pltpu.async_copy(
                    rows_v.at[p, pl.ds(j * CHUNK, CHUNK)],
                    z_ref.at[idx_v.at[s * SPC + j]],
                    ssem,
                )
                for j in range(SPC)
            ]
        for s in range(max(n_super - 2, 0), n_super):
            for d in s_descs[s]:
                d.wait()

    return _scatter


_sc_gather_q = _make_sc_gather(NQ, PER_WQ, CHUNKS_WQ, N_SUPERQ)
_sc_scatter_q = _make_sc_scatter(PER_WQ, CHUNKS_WQ, N_SUPERQ)


def _mm_q(tdf, g, wt, brow):
    grid = (NQ // BM,)
    return pl.pallas_call(
        _mm_body,
        grid=grid,
        in_specs=[
            pl.BlockSpec(memory_space=pltpu.SMEM),
            pl.BlockSpec((BM, CP), lambda i: (i, 0)),
            pl.BlockSpec((CP, CP), lambda i: (0, 0)),
            pl.BlockSpec((1, CP), lambda i: (0, 0)),
        ],
        out_specs=pl.BlockSpec((BM, CP), lambda i: (i, 0)),
        out_shape=jax.ShapeDtypeStruct((NQ, CP), jnp.float32),
    )(tdf, g, wt, brow)


def kernel(x, indices, weight, bias, to_dense):
    pos = indices[:, 0] * (H * W) + indices[:, 1] * W + indices[:, 2]
    pos2d = pos.reshape(N_PTS // CHUNK, CHUNK)
    rows_q = (N_PTS // CHUNK) // QSPLIT

    z2d = _transpose_fwd(x)

    tdf = jnp.where(to_dense, jnp.float32(1.0), jnp.float32(0.0)).reshape(1)
    wt_pad = jnp.zeros((CP, CP), jnp.float32).at[:C, :C].set(weight.T)
    b_pad = jnp.zeros((1, CP), jnp.float32).at[:, :C].set(bias.reshape(1, C))

    pos_q = [pos2d[q * rows_q:(q + 1) * rows_q] for q in range(QSPLIT)]
    g_q = [_sc_gather_q(z2d, pos_q[q]) for q in range(QSPLIT)]
    y_q = [_mm_q(tdf, g_q[q], wt_pad, b_pad) for q in range(QSPLIT)]

    z_ref = jax.new_ref(z2d)
    for q in range(QSPLIT):
        _sc_scatter_q(y_q[q], pos_q[q], z_ref)
    return _transpose_bwd(z_ref[...])
